# baseline (speedup 1.0000x reference)
import functools

import jax
import jax.numpy as jnp
from jax import lax
from jax.experimental import pallas as pl
from jax.experimental.pallas import tpu as pltpu
from jax.experimental.pallas import tpu_sc as plsc

_N = 2 * 8192 * 2048
_NW = 32
_E = _N // _NW
_CH = 16384
_NCH = _E // _CH
_CW = _CH // 4
_GRP = _CH // 64

_mesh = plsc.VectorSubcoreMesh(core_axis_name="c", subcore_axis_name="s")


@functools.partial(
    pl.kernel,
    mesh=_mesh,
    compiler_params=pltpu.CompilerParams(needs_layout_passes=False),
    out_type=[
        jax.ShapeDtypeStruct((_NW, 16), jnp.float32),
        jax.ShapeDtypeStruct((_NW, 16), jnp.float32),
    ],
    scratch_types=[
        pltpu.VMEM((_CH,), jnp.float32),
        pltpu.VMEM((_CH,), jnp.float32),
        pltpu.VMEM((_CW,), jnp.int32),
        pltpu.VMEM((16,), jnp.float32),
        pltpu.VMEM((16,), jnp.float32),
        pltpu.SemaphoreType.DMA,
    ],
)
def _sc_partials(p_hbm, t_hbm, m_hbm, out_sq, out_cnt,
                 p_v, t_v, m_v, sq_v, cnt_v, sem):
    cid = lax.axis_index("c")
    sid = lax.axis_index("s")
    wid = sid * 2 + cid
    base = wid * _E
    basew = wid * (_E // 4)

    iota4 = lax.iota(jnp.int32, 16) * 4
    zero_f = jnp.zeros((16,), jnp.float32)
    zero_i = jnp.zeros((16,), jnp.int32)

    def chunk_body(c, carry):
        sq_t, cnt_t = carry
        cp_p = pltpu.make_async_copy(
            p_hbm.at[pl.ds(base + c * _CH, _CH)], p_v, sem)
        cp_t = pltpu.make_async_copy(
            t_hbm.at[pl.ds(base + c * _CH, _CH)], t_v, sem)
        cp_m = pltpu.make_async_copy(
            m_hbm.at[pl.ds(basew + c * _CW, _CW)], m_v, sem)
        cp_p.start()
        cp_t.start()
        cp_m.start()
        cp_p.wait()
        cp_t.wait()
        cp_m.wait()

        def grp_body(g, gcarry):
            sq, cnt = gcarry
            mw = m_v[pl.ds(g * 16, 16)]
            cnt = cnt + lax.shift_right_logical(mw * 0x01010101, 24)
            ebase = g * 64
            for k in range(4):
                mk = lax.shift_right_logical(mw, 8 * k) & 1
                mf = mk.astype(jnp.float32)
                idx = iota4 + (ebase + k)
                pk = plsc.load_gather(p_v, [idx])
                tk = plsc.load_gather(t_v, [idx])
                d = pk - tk
                sq = sq + mf * (d * d)
            return sq, cnt

        sq_c, cnt_c = lax.fori_loop(0, _GRP, grp_body, (zero_f, zero_i))
        return sq_t + sq_c, cnt_t + cnt_c.astype(jnp.float32)

    sq_tot, cnt_tot = lax.fori_loop(0, _NCH, chunk_body, (zero_f, zero_f))

    sq_v[...] = sq_tot
    cnt_v[...] = cnt_tot
    pltpu.sync_copy(sq_v, out_sq.at[wid])
    pltpu.sync_copy(cnt_v, out_cnt.at[wid])


def _fin_body(sq_ref, cnt_ref, o_ref):
    o_ref[0] = jnp.sqrt(jnp.sum(sq_ref[...]) / jnp.sum(cnt_ref[...]))


def kernel(y_pred, y_true, mask):
    p = y_pred.reshape(_N)
    t = y_true.reshape(_N)
    m32 = jax.lax.bitcast_convert_type(
        mask.view(jnp.int8).reshape(_N // 4, 4), jnp.int32)
    sq, cnt = _sc_partials(p, t, m32)
    out = pl.pallas_call(
        _fin_body,
        out_specs=pl.BlockSpec(memory_space=pltpu.SMEM),
        out_shape=jax.ShapeDtypeStruct((1,), jnp.float32),
    )(sq, cnt)
    return out[0]


# speedup vs baseline: 1.0049x; 1.0049x over previous
import functools

import jax
import jax.numpy as jnp
from jax import lax
from jax.experimental import pallas as pl
from jax.experimental.pallas import tpu as pltpu
from jax.experimental.pallas import tpu_sc as plsc

_N = 2 * 8192 * 2048
_NW = 32
_E = _N // _NW
_CH = 16384
_NCH = _E // _CH
_CW = _CH // 4
_GRP = _CH // 64

_mesh = plsc.VectorSubcoreMesh(core_axis_name="c", subcore_axis_name="s")


@functools.partial(
    pl.kernel,
    mesh=_mesh,
    compiler_params=pltpu.CompilerParams(needs_layout_passes=False),
    out_type=[
        jax.ShapeDtypeStruct((_NW, 16), jnp.float32),
        jax.ShapeDtypeStruct((_NW, 16), jnp.float32),
    ],
    scratch_types=[
        pltpu.VMEM((_CH,), jnp.float32),
        pltpu.VMEM((_CH,), jnp.float32),
        pltpu.VMEM((_CW,), jnp.int32),
        pltpu.VMEM((16,), jnp.float32),
        pltpu.VMEM((16,), jnp.float32),
        pltpu.SemaphoreType.DMA,
    ],
)
def _sc_partials(p_hbm, t_hbm, m_hbm, out_sq, out_cnt,
                 p_v, t_v, m_v, sq_v, cnt_v, sem):
    cid = lax.axis_index("c")
    sid = lax.axis_index("s")
    wid = sid * 2 + cid
    base = wid * _E
    basew = wid * (_E // 4)

    iota = lax.iota(jnp.int32, 16)
    l4 = lax.shift_right_logical(iota, 2)
    sh = (iota & 3) * 8
    zero_f = jnp.zeros((16,), jnp.float32)

    def chunk_body(c, carry):
        cp_p = pltpu.make_async_copy(
            p_hbm.at[pl.ds(base + c * _CH, _CH)], p_v, sem)
        cp_t = pltpu.make_async_copy(
            t_hbm.at[pl.ds(base + c * _CH, _CH)], t_v, sem)
        cp_m = pltpu.make_async_copy(
            m_hbm.at[pl.ds(basew + c * _CW, _CW)], m_v, sem)
        cp_p.start()
        cp_t.start()
        cp_m.start()
        cp_p.wait()
        cp_t.wait()
        cp_m.wait()

        def grp_body(g, gcarry):
            new = []
            wbase = g * 16
            ebase = g * 64
            for v in range(4):
                mw = plsc.load_gather(m_v, [l4 + (wbase + 4 * v)])
                mf = (lax.shift_right_logical(mw, sh) & 1).astype(jnp.float32)
                pv = p_v[pl.ds(ebase + 16 * v, 16)]
                tv = t_v[pl.ds(ebase + 16 * v, 16)]
                d = pv - tv
                new.append(gcarry[v] + mf * (d * d))
                new.append(gcarry[4 + v] + mf)
            return (new[0], new[2], new[4], new[6],
                    new[1], new[3], new[5], new[7])

        accs = lax.fori_loop(0, _GRP, grp_body, (zero_f,) * 8)
        sq_c = (accs[0] + accs[1]) + (accs[2] + accs[3])
        cnt_c = (accs[4] + accs[5]) + (accs[6] + accs[7])
        sq_t, cnt_t = carry
        return sq_t + sq_c, cnt_t + cnt_c

    sq_tot, cnt_tot = lax.fori_loop(0, _NCH, chunk_body, (zero_f, zero_f))

    sq_v[...] = sq_tot
    cnt_v[...] = cnt_tot
    pltpu.sync_copy(sq_v, out_sq.at[wid])
    pltpu.sync_copy(cnt_v, out_cnt.at[wid])


def _fin_body(sq_ref, cnt_ref, o_ref):
    o_ref[0] = jnp.sqrt(jnp.sum(sq_ref[...]) / jnp.sum(cnt_ref[...]))


def kernel(y_pred, y_true, mask):
    p = y_pred.reshape(_N)
    t = y_true.reshape(_N)
    m32 = jax.lax.bitcast_convert_type(
        mask.view(jnp.int8).reshape(_N // 4, 4), jnp.int32)
    sq, cnt = _sc_partials(p, t, m32)
    out = pl.pallas_call(
        _fin_body,
        out_specs=pl.BlockSpec(memory_space=pltpu.SMEM),
        out_shape=jax.ShapeDtypeStruct((1,), jnp.float32),
    )(sq, cnt)
    return out[0]


# speedup vs baseline: 16.4362x; 16.3565x over previous
import functools

import jax
import jax.numpy as jnp
from jax import lax
from jax.experimental import pallas as pl
from jax.experimental.pallas import tpu as pltpu
from jax.experimental.pallas import tpu_sc as plsc

_ROWS = 16384
_COLS = 2048
_N = _ROWS * _COLS

_SC_ROWS = 512
_NSC = _SC_ROWS * _COLS
_NW = 32
_E = _NSC // _NW
_EW = _E // 4
_GRP = _E // 64

_BR = 512
_TC_BLK0 = _SC_ROWS // _BR
_TC_GRID = (_ROWS - _SC_ROWS) // _BR

_mesh = plsc.VectorSubcoreMesh(core_axis_name="c", subcore_axis_name="s")


@functools.partial(
    pl.kernel,
    mesh=_mesh,
    compiler_params=pltpu.CompilerParams(needs_layout_passes=False),
    out_type=[
        jax.ShapeDtypeStruct((_NW, 16), jnp.float32),
        jax.ShapeDtypeStruct((_NW, 16), jnp.float32),
    ],
    scratch_types=[
        pltpu.VMEM((_E,), jnp.float32),
        pltpu.VMEM((_E,), jnp.float32),
        pltpu.VMEM((_EW,), jnp.int32),
        pltpu.VMEM((16,), jnp.float32),
        pltpu.VMEM((16,), jnp.float32),
        pltpu.SemaphoreType.DMA,
    ],
)
def _sc_partials(p_hbm, t_hbm, m_hbm, out_sq, out_cnt,
                 p_v, t_v, m_v, sq_v, cnt_v, sem):
    cid = lax.axis_index("c")
    sid = lax.axis_index("s")
    wid = sid * 2 + cid
    base = wid * _E
    basew = wid * _EW

    cp_p = pltpu.make_async_copy(p_hbm.at[pl.ds(base, _E)], p_v, sem)
    cp_t = pltpu.make_async_copy(t_hbm.at[pl.ds(base, _E)], t_v, sem)
    cp_m = pltpu.make_async_copy(m_hbm.at[pl.ds(basew, _EW)], m_v, sem)
    cp_p.start()
    cp_t.start()
    cp_m.start()
    cp_p.wait()
    cp_t.wait()
    cp_m.wait()

    iota = lax.iota(jnp.int32, 16)
    l4 = lax.shift_right_logical(iota, 2)
    sh = (iota & 3) * 8
    zero_f = jnp.zeros((16,), jnp.float32)

    def grp_body(g, gcarry):
        new = []
        wbase = g * 16
        ebase = g * 64
        for v in range(4):
            mw = plsc.load_gather(m_v, [l4 + (wbase + 4 * v)])
            mf = (lax.shift_right_logical(mw, sh) & 1).astype(jnp.float32)
            pv = p_v[pl.ds(ebase + 16 * v, 16)]
            tv = t_v[pl.ds(ebase + 16 * v, 16)]
            d = pv - tv
            new.append(gcarry[v] + mf * (d * d))
            new.append(gcarry[4 + v] + mf)
        return (new[0], new[2], new[4], new[6],
                new[1], new[3], new[5], new[7])

    accs = lax.fori_loop(0, _GRP, grp_body, (zero_f,) * 8)
    sq_v[...] = (accs[0] + accs[1]) + (accs[2] + accs[3])
    cnt_v[...] = (accs[4] + accs[5]) + (accs[6] + accs[7])
    pltpu.sync_copy(sq_v, out_sq.at[wid])
    pltpu.sync_copy(cnt_v, out_cnt.at[wid])


def _tc_body(p_ref, t_ref, m_ref, o_ref, acc_ref):
    i = pl.program_id(0)

    @pl.when(i == 0)
    def _init():
        acc_ref[0] = 0.0
        acc_ref[1] = 0.0

    m = m_ref[...].astype(jnp.float32)
    d = p_ref[...] - t_ref[...]
    acc_ref[0] += jnp.sum(d * d * m)
    acc_ref[1] += jnp.sum(m)

    @pl.when(i == pl.num_programs(0) - 1)
    def _fin():
        o_ref[0] = acc_ref[0]
        o_ref[1] = acc_ref[1]


def _fin_body(tc_ref, sq_ref, cnt_ref, o_ref):
    total = tc_ref[0] + jnp.sum(sq_ref[...])
    count = tc_ref[1] + jnp.sum(cnt_ref[...])
    o_ref[0] = jnp.sqrt(total / count)


def kernel(y_pred, y_true, mask):
    p2 = y_pred.reshape(_ROWS, _COLS)
    t2 = y_true.reshape(_ROWS, _COLS)
    m8 = mask.view(jnp.int8).reshape(_ROWS, _COLS)

    pf = y_pred.reshape(_N)
    tf = y_true.reshape(_N)
    m32_sc = jax.lax.bitcast_convert_type(
        mask.view(jnp.int8).reshape(_N)[: _NSC].reshape(_NSC // 4, 4),
        jnp.int32,
    )

    sc_sq, sc_cnt = _sc_partials(pf, tf, m32_sc)

    tc_part = pl.pallas_call(
        _tc_body,
        grid=(_TC_GRID,),
        in_specs=[
            pl.BlockSpec((_BR, _COLS), lambda i: (i + _TC_BLK0, 0)),
            pl.BlockSpec((_BR, _COLS), lambda i: (i + _TC_BLK0, 0)),
            pl.BlockSpec((_BR, _COLS), lambda i: (i + _TC_BLK0, 0)),
        ],
        out_specs=pl.BlockSpec(memory_space=pltpu.SMEM),
        out_shape=jax.ShapeDtypeStruct((2,), jnp.float32),
        scratch_shapes=[pltpu.SMEM((2,), jnp.float32)],
        compiler_params=pltpu.CompilerParams(
            dimension_semantics=("arbitrary",),
        ),
    )(p2, t2, m8)

    out = pl.pallas_call(
        _fin_body,
        in_specs=[
            pl.BlockSpec(memory_space=pltpu.SMEM),
            pl.BlockSpec(),
            pl.BlockSpec(),
        ],
        out_specs=pl.BlockSpec(memory_space=pltpu.SMEM),
        out_shape=jax.ShapeDtypeStruct((1,), jnp.float32),
    )(tc_part, sc_sq, sc_cnt)
    return out[0]
